# Initial kernel scaffold; baseline (speedup 1.0000x reference)
import jax
import jax.numpy as jnp
from jax.experimental import pallas as pl


def kernel(feature0, feature1, b_idxes, i_idxes, j_idxes):
    raise NotImplementedError("write your pallas kernel here")


# speedup vs baseline: 1.9039x; 1.9039x over previous
import functools
import jax
import jax.numpy as jnp
from jax import lax
from jax.experimental import pallas as pl
from jax.experimental.pallas import tpu as pltpu
from jax.experimental.pallas import tpu_sc as plsc

_W_SIZE = 8
_STRIDE = 4
_PAD = 2
_EXTRA = 2

_B, _C, _H, _W = 2, 128, 192, 192
_GRID = (_H + 2 * _PAD - _W_SIZE) // _STRIDE + 1
_M = 3000

_H0 = _H + 2 * _PAD
_T0 = _H0 // 4
_ROWS0 = _W_SIZE * 2
_H1 = _H + 2 * (_PAD + _EXTRA)
_T1 = _H1 // 4
_K1 = _W_SIZE + 2 * _EXTRA
_ROWS1 = _K1 * 3

_NW = 32
_MPW = 96
_CH0 = 4
_CH1 = 2


def _gather_kernel(f0t, f1t, idx0, idx1, out0, out1, i0_v, i1_v, b0_v, b1_v,
                   sem):
    wid = lax.axis_index("c") * 16 + lax.axis_index("s")
    last = wid == _NW - 1

    n0 = jnp.where(last, (_M - (_NW - 1) * _MPW) // _CH0, _MPW // _CH0)
    base0 = wid * (_MPW * _ROWS0)

    def body0(g, carry):
        off = base0 + g * (_CH0 * _ROWS0)
        pltpu.sync_copy(idx0.at[pl.ds(off, _CH0 * _ROWS0)], i0_v)
        pltpu.async_copy(f0t.at[i0_v], b0_v, sem).wait()
        pltpu.sync_copy(b0_v, out0.at[pl.ds(off, _CH0 * _ROWS0)])
        return carry

    lax.fori_loop(0, n0, body0, 0)

    n1 = jnp.where(last, (_M - (_NW - 1) * _MPW) // _CH1, _MPW // _CH1)
    base1 = wid * (_MPW * _ROWS1)

    def body1(g, carry):
        off = base1 + g * (_CH1 * _ROWS1)
        pltpu.sync_copy(idx1.at[pl.ds(off, _CH1 * _ROWS1)], i1_v)
        pltpu.async_copy(f1t.at[i1_v], b1_v, sem).wait()
        pltpu.sync_copy(b1_v, out1.at[pl.ds(off, _CH1 * _ROWS1)])
        return carry

    lax.fori_loop(0, n1, body1, 0)


@jax.jit
def kernel(feature0, feature1, b_idxes, i_idxes, j_idxes):
    f0t = jnp.pad(jnp.transpose(feature0, (0, 2, 3, 1)),
                  ((0, 0), (_PAD, _PAD), (_PAD, _PAD), (0, 0)))
    f0t = f0t.reshape(_B * _H0 * _T0, 4 * _C)
    f1t = jnp.pad(jnp.transpose(feature1, (0, 2, 3, 1)),
                  ((0, 0), (_PAD + _EXTRA, _PAD + _EXTRA),
                   (_PAD + _EXTRA, _PAD + _EXTRA), (0, 0)))
    f1t = f1t.reshape(_B * _H1 * _T1, 4 * _C)

    b = b_idxes.astype(jnp.int32)
    i = i_idxes.astype(jnp.int32)
    j = j_idxes.astype(jnp.int32)

    r0 = (i // _GRID) * _STRIDE
    c0 = (i % _GRID) * _STRIDE
    ki0 = jnp.arange(_W_SIZE, dtype=jnp.int32)
    s0 = jnp.arange(2, dtype=jnp.int32)
    idx0 = (b * (_H0 * _T0))[:, None, None] \
        + ((r0[:, None] + ki0[None, :]) * _T0)[:, :, None] \
        + (c0 // 4)[:, None, None] + s0[None, None, :]
    idx0 = idx0.reshape(_M * _ROWS0)

    r1 = (j // _GRID) * _STRIDE
    c1 = (j % _GRID) * _STRIDE
    ki1 = jnp.arange(_K1, dtype=jnp.int32)
    s1 = jnp.arange(3, dtype=jnp.int32)
    idx1 = (b * (_H1 * _T1))[:, None, None] \
        + ((r1[:, None] + ki1[None, :]) * _T1)[:, :, None] \
        + (c1 // 4)[:, None, None] + s1[None, None, :]
    idx1 = idx1.reshape(_M * _ROWS1)

    mesh = plsc.VectorSubcoreMesh(core_axis_name="c", subcore_axis_name="s")
    out0, out1 = pl.kernel(
        _gather_kernel,
        mesh=mesh,
        out_type=[
            jax.ShapeDtypeStruct((_M * _ROWS0, 4 * _C), jnp.float32),
            jax.ShapeDtypeStruct((_M * _ROWS1, 4 * _C), jnp.float32),
        ],
        scratch_types=[
            pltpu.VMEM((_CH0 * _ROWS0,), jnp.int32),
            pltpu.VMEM((_CH1 * _ROWS1,), jnp.int32),
            pltpu.VMEM((_CH0 * _ROWS0, 4 * _C), jnp.float32),
            pltpu.VMEM((_CH1 * _ROWS1, 4 * _C), jnp.float32),
            pltpu.SemaphoreType.DMA,
        ],
    )(f0t, f1t, idx0, idx1)

    fine0 = out0.reshape(_M, _W_SIZE * _W_SIZE, _C)
    fine1 = out1.reshape(_M, _K1 * _K1, _C)
    return (fine0, fine1)


# speedup vs baseline: 2.4344x; 1.2786x over previous
import functools
import jax
import jax.numpy as jnp
from jax import lax
from jax.experimental import pallas as pl
from jax.experimental.pallas import tpu as pltpu
from jax.experimental.pallas import tpu_sc as plsc

_W_SIZE = 8
_STRIDE = 4
_PAD = 2
_EXTRA = 2

_B, _C, _H, _W = 2, 128, 192, 192
_GRID = (_H + 2 * _PAD - _W_SIZE) // _STRIDE + 1
_M = 3000

_H0 = _H + 2 * _PAD
_PIX0 = _W_SIZE * _W_SIZE
_H1 = _H + 2 * (_PAD + _EXTRA)
_K1 = _W_SIZE + 2 * _EXTRA
_PIX1 = _K1 * _K1

_NW = 32
_MPW = 96
_CH0 = 2 * _PIX0
_CH1 = _PIX1 // 2


def _gather_kernel(f0t, f1t, idx0, idx1, out0, out1, i0_v, i1_v, b0_v, b1_v,
                   sem):
    wid = lax.axis_index("c") * 16 + lax.axis_index("s")
    last = wid == _NW - 1

    n0 = jnp.where(last, (_M - (_NW - 1) * _MPW) // 2, _MPW // 2)
    base0 = wid * (_MPW * _PIX0)

    def body0(g, carry):
        off = base0 + g * _CH0
        pltpu.sync_copy(idx0.at[pl.ds(off, _CH0)], i0_v)
        pltpu.async_copy(f0t.at[i0_v], b0_v, sem).wait()
        pltpu.sync_copy(b0_v, out0.at[pl.ds(off, _CH0)])
        return carry

    lax.fori_loop(0, n0, body0, 0)

    n1 = jnp.where(last, (_M - (_NW - 1) * _MPW) * 2, _MPW * 2)
    base1 = wid * (_MPW * _PIX1)

    def body1(g, carry):
        off = base1 + g * _CH1
        pltpu.sync_copy(idx1.at[pl.ds(off, _CH1)], i1_v)
        pltpu.async_copy(f1t.at[i1_v], b1_v, sem).wait()
        pltpu.sync_copy(b1_v, out1.at[pl.ds(off, _CH1)])
        return carry

    lax.fori_loop(0, n1, body1, 0)


@jax.jit
def kernel(feature0, feature1, b_idxes, i_idxes, j_idxes):
    f0t = jnp.pad(jnp.transpose(feature0, (0, 2, 3, 1)),
                  ((0, 0), (_PAD, _PAD), (_PAD, _PAD), (0, 0)))
    f0t = f0t.reshape(_B * _H0 * _H0, _C)
    f1t = jnp.pad(jnp.transpose(feature1, (0, 2, 3, 1)),
                  ((0, 0), (_PAD + _EXTRA, _PAD + _EXTRA),
                   (_PAD + _EXTRA, _PAD + _EXTRA), (0, 0)))
    f1t = f1t.reshape(_B * _H1 * _H1, _C)

    b = b_idxes.astype(jnp.int32)
    i = i_idxes.astype(jnp.int32)
    j = j_idxes.astype(jnp.int32)

    r0 = (i // _GRID) * _STRIDE
    c0 = (i % _GRID) * _STRIDE
    p0 = jnp.arange(_PIX0, dtype=jnp.int32)
    idx0 = b[:, None] * (_H0 * _H0) \
        + (r0[:, None] + p0[None, :] // _W_SIZE) * _H0 \
        + c0[:, None] + p0[None, :] % _W_SIZE
    idx0 = idx0.reshape(_M * _PIX0)

    r1 = (j // _GRID) * _STRIDE
    c1 = (j % _GRID) * _STRIDE
    p1 = jnp.arange(_PIX1, dtype=jnp.int32)
    idx1 = b[:, None] * (_H1 * _H1) \
        + (r1[:, None] + p1[None, :] // _K1) * _H1 \
        + c1[:, None] + p1[None, :] % _K1
    idx1 = idx1.reshape(_M * _PIX1)

    mesh = plsc.VectorSubcoreMesh(core_axis_name="c", subcore_axis_name="s")
    out0, out1 = pl.kernel(
        _gather_kernel,
        mesh=mesh,
        out_type=[
            jax.ShapeDtypeStruct((_M * _PIX0, _C), jnp.float32),
            jax.ShapeDtypeStruct((_M * _PIX1, _C), jnp.float32),
        ],
        scratch_types=[
            pltpu.VMEM((_CH0,), jnp.int32),
            pltpu.VMEM((_CH1,), jnp.int32),
            pltpu.VMEM((_CH0, _C), jnp.float32),
            pltpu.VMEM((_CH1, _C), jnp.float32),
            pltpu.SemaphoreType.DMA,
        ],
    )(f0t, f1t, idx0, idx1)

    fine0 = out0.reshape(_M, _PIX0, _C)
    fine1 = out1.reshape(_M, _PIX1, _C)
    return (fine0, fine1)


# speedup vs baseline: 3.3467x; 1.3748x over previous
import functools
import jax
import jax.numpy as jnp
from jax import lax
from jax.experimental import pallas as pl
from jax.experimental.pallas import tpu as pltpu
from jax.experimental.pallas import tpu_sc as plsc

_W_SIZE = 8
_STRIDE = 4
_PAD = 2
_EXTRA = 2

_B, _C, _H, _W = 2, 128, 192, 192
_GRID = (_H + 2 * _PAD - _W_SIZE) // _STRIDE + 1
_M = 3000

_H0 = _H + 2 * _PAD
_PIX0 = _W_SIZE * _W_SIZE
_H1 = _H + 2 * (_PAD + _EXTRA)
_K1 = _W_SIZE + 2 * _EXTRA
_PIX1 = _K1 * _K1

_NW = 32
_NB0 = _M * _PIX0 // 128
_NB1 = _M * _PIX1 // 128
_BPW0 = 48
_BPW1 = 108
_STRIDE1 = 112


def _gather_kernel(f0t, f1t, idx0, idx1, out0, out1, i0_v, i1_v, b0_v, b1_v,
                   sem):
    wid = lax.axis_index("c") * 16 + lax.axis_index("s")
    last = wid == _NW - 1

    nb0 = jnp.where(last, _NB0 - (_NW - 1) * _BPW0, _BPW0)
    base0 = wid * _BPW0
    pltpu.sync_copy(idx0.at[pl.ds(base0, _BPW0)], i0_v)

    def body0(g, carry):
        blk = g
        pltpu.async_copy(f0t.at[i0_v.at[blk]], b0_v, sem).wait()
        pltpu.sync_copy(b0_v, out0.at[base0 + blk])
        return carry

    lax.fori_loop(0, nb0, body0, 0)

    nb1 = jnp.where(last, _NB1 - (_NW - 1) * _BPW1, _BPW1)
    base1 = wid * _BPW1
    pltpu.sync_copy(idx1.at[pl.ds(wid * _STRIDE1, _STRIDE1)], i1_v)

    def body1(g, carry):
        blk = g
        pltpu.async_copy(f1t.at[i1_v.at[blk]], b1_v, sem).wait()
        pltpu.sync_copy(b1_v, out1.at[base1 + blk])
        return carry

    lax.fori_loop(0, nb1, body1, 0)


@jax.jit
def kernel(feature0, feature1, b_idxes, i_idxes, j_idxes):
    f0t = jnp.pad(jnp.transpose(feature0, (0, 2, 3, 1)),
                  ((0, 0), (_PAD, _PAD), (_PAD, _PAD), (0, 0)))
    f0t = f0t.reshape(_B * _H0 * _H0, _C)
    f1t = jnp.pad(jnp.transpose(feature1, (0, 2, 3, 1)),
                  ((0, 0), (_PAD + _EXTRA, _PAD + _EXTRA),
                   (_PAD + _EXTRA, _PAD + _EXTRA), (0, 0)))
    f1t = f1t.reshape(_B * _H1 * _H1, _C)

    b = b_idxes.astype(jnp.int32)
    i = i_idxes.astype(jnp.int32)
    j = j_idxes.astype(jnp.int32)

    r0 = (i // _GRID) * _STRIDE
    c0 = (i % _GRID) * _STRIDE
    p0 = jnp.arange(_PIX0, dtype=jnp.int32)
    idx0 = b[:, None] * (_H0 * _H0) \
        + (r0[:, None] + p0[None, :] // _W_SIZE) * _H0 \
        + c0[:, None] + p0[None, :] % _W_SIZE
    idx0 = idx0.reshape(_NB0, 128)
    idx0 = jnp.pad(idx0, ((0, _NW * _BPW0 - _NB0), (0, 0)))

    r1 = (j // _GRID) * _STRIDE
    c1 = (j % _GRID) * _STRIDE
    p1 = jnp.arange(_PIX1, dtype=jnp.int32)
    idx1 = b[:, None] * (_H1 * _H1) \
        + (r1[:, None] + p1[None, :] // _K1) * _H1 \
        + c1[:, None] + p1[None, :] % _K1
    idx1 = idx1.reshape(_NB1, 128)
    idx1 = jnp.pad(idx1, ((0, _NW * _BPW1 - _NB1), (0, 0)))
    idx1 = jnp.pad(idx1.reshape(_NW, _BPW1, 128),
                   ((0, 0), (0, _STRIDE1 - _BPW1), (0, 0)))
    idx1 = idx1.reshape(_NW * _STRIDE1, 128)

    mesh = plsc.VectorSubcoreMesh(core_axis_name="c", subcore_axis_name="s")
    out0, out1 = pl.kernel(
        _gather_kernel,
        mesh=mesh,
        out_type=[
            jax.ShapeDtypeStruct((_NB0, 128, _C), jnp.float32),
            jax.ShapeDtypeStruct((_NB1, 128, _C), jnp.float32),
        ],
        scratch_types=[
            pltpu.VMEM((_BPW0, 128), jnp.int32),
            pltpu.VMEM((_STRIDE1, 128), jnp.int32),
            pltpu.VMEM((128, _C), jnp.float32),
            pltpu.VMEM((128, _C), jnp.float32),
            pltpu.SemaphoreType.DMA,
        ],
    )(f0t, f1t, idx0, idx1)

    fine0 = out0.reshape(_M, _PIX0, _C)
    fine1 = out1.reshape(_M, _PIX1, _C)
    return (fine0, fine1)


# speedup vs baseline: 3.6563x; 1.0925x over previous
import functools
import jax
import jax.numpy as jnp
from jax import lax
from jax.experimental import pallas as pl
from jax.experimental.pallas import tpu as pltpu
from jax.experimental.pallas import tpu_sc as plsc

_W_SIZE = 8
_STRIDE = 4
_PAD = 2
_EXTRA = 2

_B, _C, _H, _W = 2, 128, 192, 192
_GRID = (_H + 2 * _PAD - _W_SIZE) // _STRIDE + 1
_M = 3000

_H0 = _H + 2 * _PAD
_W0P = 200
_PIX0 = _W_SIZE * _W_SIZE
_H1 = _H + 2 * (_PAD + _EXTRA)
_K1 = _W_SIZE + 2 * _EXTRA
_PIX1 = _K1 * _K1

_NW = 32
_NB0 = _M * _PIX0 // 128
_NB1 = _M * _PIX1 // 128
_BPW0 = 48
_BPW1 = 108
_STRIDE1 = 112


def _gather_kernel(f0t, f1t, idx0, idx1, out0, out1, i0_v, i1_v, b0_v, b1_v,
                   sem):
    wid = lax.axis_index("c") * 16 + lax.axis_index("s")
    last = wid == _NW - 1

    nb0 = jnp.where(last, _NB0 - (_NW - 1) * _BPW0, _BPW0)
    base0 = wid * _BPW0
    pltpu.sync_copy(idx0.at[pl.ds(base0, _BPW0)], i0_v)

    def body0(g, carry):
        blk = g
        pltpu.async_copy(f0t.at[i0_v.at[blk]], b0_v, sem).wait()
        pltpu.sync_copy(b0_v, out0.at[base0 + blk])
        return carry

    lax.fori_loop(0, nb0, body0, 0)

    nb1 = jnp.where(last, _NB1 - (_NW - 1) * _BPW1, _BPW1)
    base1 = wid * _BPW1
    pltpu.sync_copy(idx1.at[pl.ds(wid * _STRIDE1, _STRIDE1)], i1_v)

    def body1(g, carry):
        blk = g
        pltpu.async_copy(f1t.at[i1_v.at[blk]], b1_v, sem).wait()
        pltpu.sync_copy(b1_v, out1.at[base1 + blk])
        return carry

    lax.fori_loop(0, nb1, body1, 0)


@jax.jit
def kernel(feature0, feature1, b_idxes, i_idxes, j_idxes):
    f0t = jnp.pad(jnp.transpose(feature0, (0, 2, 3, 1)),
                  ((0, 0), (_PAD, _PAD), (_PAD, _W0P - _W - _PAD), (0, 0)))
    f0t = f0t.reshape(_B * _H0 * _W0P, _C)
    f1t = jnp.pad(jnp.transpose(feature1, (0, 2, 3, 1)),
                  ((0, 0), (_PAD + _EXTRA, _PAD + _EXTRA),
                   (_PAD + _EXTRA, _PAD + _EXTRA), (0, 0)))
    f1t = f1t.reshape(_B * _H1 * _H1, _C)

    b = b_idxes.astype(jnp.int32)
    i = i_idxes.astype(jnp.int32)
    j = j_idxes.astype(jnp.int32)

    r0 = (i // _GRID) * _STRIDE
    c0 = (i % _GRID) * _STRIDE
    p0 = jnp.arange(_PIX0, dtype=jnp.int32)
    idx0 = b[:, None] * (_H0 * _W0P) \
        + (r0[:, None] + p0[None, :] // _W_SIZE) * _W0P \
        + c0[:, None] + p0[None, :] % _W_SIZE
    idx0 = idx0.reshape(_NB0, 128)
    idx0 = jnp.pad(idx0, ((0, _NW * _BPW0 - _NB0), (0, 0)))

    r1 = (j // _GRID) * _STRIDE
    c1 = (j % _GRID) * _STRIDE
    p1 = jnp.arange(_PIX1, dtype=jnp.int32)
    idx1 = b[:, None] * (_H1 * _H1) \
        + (r1[:, None] + p1[None, :] // _K1) * _H1 \
        + c1[:, None] + p1[None, :] % _K1
    idx1 = idx1.reshape(_NB1, 128)
    idx1 = jnp.pad(idx1, ((0, _NW * _BPW1 - _NB1), (0, 0)))
    idx1 = jnp.pad(idx1.reshape(_NW, _BPW1, 128),
                   ((0, 0), (0, _STRIDE1 - _BPW1), (0, 0)))
    idx1 = idx1.reshape(_NW * _STRIDE1, 128)

    mesh = plsc.VectorSubcoreMesh(core_axis_name="c", subcore_axis_name="s")
    out0, out1 = pl.kernel(
        _gather_kernel,
        mesh=mesh,
        out_type=[
            jax.ShapeDtypeStruct((_NB0, 128, _C), jnp.float32),
            jax.ShapeDtypeStruct((_NB1, 128, _C), jnp.float32),
        ],
        scratch_types=[
            pltpu.VMEM((_BPW0, 128), jnp.int32),
            pltpu.VMEM((_STRIDE1, 128), jnp.int32),
            pltpu.VMEM((128, _C), jnp.float32),
            pltpu.VMEM((128, _C), jnp.float32),
            pltpu.SemaphoreType.DMA,
        ],
    )(f0t, f1t, idx0, idx1)

    fine0 = out0.reshape(_M, _PIX0, _C)
    fine1 = out1.reshape(_M, _PIX1, _C)
    return (fine0, fine1)


# speedup vs baseline: 4.4495x; 1.2169x over previous
import functools
import jax
import jax.numpy as jnp
from jax import lax
from jax.experimental import pallas as pl
from jax.experimental.pallas import tpu as pltpu
from jax.experimental.pallas import tpu_sc as plsc

_W_SIZE = 8
_STRIDE = 4
_PAD = 2
_EXTRA = 2

_B, _C, _H, _W = 2, 128, 192, 192
_GRID = (_H + 2 * _PAD - _W_SIZE) // _STRIDE + 1
_M = 3000

_H0 = _H + 2 * _PAD
_W0P = 200
_PIX0 = _W_SIZE * _W_SIZE
_H1 = _H + 2 * (_PAD + _EXTRA)
_K1 = _W_SIZE + 2 * _EXTRA
_PIX1 = _K1 * _K1

_NW = 32
_NB0 = _M * _PIX0 // 128
_NB1 = _M * _PIX1 // 128
_BPW0 = 48
_BPW1 = 108
_STRIDE1 = 112


def _gather_kernel(f0t, f1t, idx0, idx1, out0, out1, i0_v, i1_v, b_a, b_b,
                   sem_a, sem_b):
    wid = lax.axis_index("c") * 16 + lax.axis_index("s")
    last = wid == _NW - 1

    def run_pass(table, idx_v, out, base, nb):
        def body(g2, carry):
            g_a = 2 * g2
            g_b = g_a + 1
            copy_a = pltpu.async_copy(table.at[idx_v.at[g_a]], b_a, sem_a)

            @pl.when(g_b < nb)
            def _():
                pltpu.async_copy(table.at[idx_v.at[g_b]], b_b, sem_b)

            copy_a.wait()
            pltpu.sync_copy(b_a, out.at[base + g_a])

            @pl.when(g_b < nb)
            def _():
                pltpu.make_async_copy(table.at[idx_v.at[g_b]], b_b,
                                      sem_b).wait()
                pltpu.sync_copy(b_b, out.at[base + g_b])

            return carry

        lax.fori_loop(0, (nb + 1) // 2, body, 0)

    nb0 = jnp.where(last, _NB0 - (_NW - 1) * _BPW0, _BPW0)
    base0 = wid * _BPW0
    pltpu.sync_copy(idx0.at[pl.ds(base0, _BPW0)], i0_v)
    run_pass(f0t, i0_v, out0, base0, nb0)

    nb1 = jnp.where(last, _NB1 - (_NW - 1) * _BPW1, _BPW1)
    base1 = wid * _BPW1
    pltpu.sync_copy(idx1.at[pl.ds(wid * _STRIDE1, _STRIDE1)], i1_v)
    run_pass(f1t, i1_v, out1, base1, nb1)


@jax.jit
def kernel(feature0, feature1, b_idxes, i_idxes, j_idxes):
    f0t = jnp.pad(jnp.transpose(feature0, (0, 2, 3, 1)),
                  ((0, 0), (_PAD, _PAD), (_PAD, _W0P - _W - _PAD), (0, 0)))
    f0t = f0t.reshape(_B * _H0 * _W0P, _C)
    f1t = jnp.pad(jnp.transpose(feature1, (0, 2, 3, 1)),
                  ((0, 0), (_PAD + _EXTRA, _PAD + _EXTRA),
                   (_PAD + _EXTRA, _PAD + _EXTRA), (0, 0)))
    f1t = f1t.reshape(_B * _H1 * _H1, _C)

    b = b_idxes.astype(jnp.int32)
    i = i_idxes.astype(jnp.int32)
    j = j_idxes.astype(jnp.int32)

    r0 = (i // _GRID) * _STRIDE
    c0 = (i % _GRID) * _STRIDE
    p0 = jnp.arange(_PIX0, dtype=jnp.int32)
    idx0 = b[:, None] * (_H0 * _W0P) \
        + (r0[:, None] + p0[None, :] // _W_SIZE) * _W0P \
        + c0[:, None] + p0[None, :] % _W_SIZE
    idx0 = idx0.reshape(_NB0, 128)
    idx0 = jnp.pad(idx0, ((0, _NW * _BPW0 - _NB0), (0, 0)))

    r1 = (j // _GRID) * _STRIDE
    c1 = (j % _GRID) * _STRIDE
    p1 = jnp.arange(_PIX1, dtype=jnp.int32)
    idx1 = b[:, None] * (_H1 * _H1) \
        + (r1[:, None] + p1[None, :] // _K1) * _H1 \
        + c1[:, None] + p1[None, :] % _K1
    idx1 = idx1.reshape(_NB1, 128)
    idx1 = jnp.pad(idx1, ((0, _NW * _BPW1 - _NB1), (0, 0)))
    idx1 = jnp.pad(idx1.reshape(_NW, _BPW1, 128),
                   ((0, 0), (0, _STRIDE1 - _BPW1), (0, 0)))
    idx1 = idx1.reshape(_NW * _STRIDE1, 128)

    mesh = plsc.VectorSubcoreMesh(core_axis_name="c", subcore_axis_name="s")
    out0, out1 = pl.kernel(
        _gather_kernel,
        mesh=mesh,
        out_type=[
            jax.ShapeDtypeStruct((_NB0, 128, _C), jnp.float32),
            jax.ShapeDtypeStruct((_NB1, 128, _C), jnp.float32),
        ],
        scratch_types=[
            pltpu.VMEM((_BPW0, 128), jnp.int32),
            pltpu.VMEM((_STRIDE1, 128), jnp.int32),
            pltpu.VMEM((128, _C), jnp.float32),
            pltpu.VMEM((128, _C), jnp.float32),
            pltpu.SemaphoreType.DMA,
            pltpu.SemaphoreType.DMA,
        ],
    )(f0t, f1t, idx0, idx1)

    fine0 = out0.reshape(_M, _PIX0, _C)
    fine1 = out1.reshape(_M, _PIX1, _C)
    return (fine0, fine1)


# speedup vs baseline: 5.0586x; 1.1369x over previous
import functools
import jax
import jax.numpy as jnp
from jax import lax
from jax.experimental import pallas as pl
from jax.experimental.pallas import tpu as pltpu
from jax.experimental.pallas import tpu_sc as plsc

_W_SIZE = 8
_STRIDE = 4
_PAD = 2
_EXTRA = 2

_B, _C, _H, _W = 2, 128, 192, 192
_GRID = (_H + 2 * _PAD - _W_SIZE) // _STRIDE + 1
_M = 3000

_H0 = _H + 2 * _PAD
_W0P = 200
_PIX0 = _W_SIZE * _W_SIZE
_H1 = _H + 2 * (_PAD + _EXTRA)
_K1 = _W_SIZE + 2 * _EXTRA
_PIX1 = _K1 * _K1

_NW = 32
_NB0 = _M * _PIX0 // 128
_NB1 = _M * _PIX1 // 128
_BPW0 = 48
_BPW1 = 108
_STRIDE1 = 112


def _gather_kernel(f0t, f1t, idx0, idx1, out0, out1, i0_v, i1_v,
                   b0, b1, b2, b3, gs0, gs1, gs2, gs3, ws0, ws1, ws2, ws3):
    bufs = (b0, b1, b2, b3)
    gsems = (gs0, gs1, gs2, gs3)
    wsems = (ws0, ws1, ws2, ws3)
    wid = lax.axis_index("c") * 16 + lax.axis_index("s")
    last = wid == _NW - 1
    nring = len(bufs)

    def run_pass(table, idx_v, out, base, nb):
        nq = nb // nring

        def body(gq, carry):
            for k in range(nring):
                g = nring * gq + k

                @pl.when(gq > 0)
                def _(k=k):
                    pltpu.make_async_copy(bufs[k], out.at[base],
                                          wsems[k]).wait()

                pltpu.async_copy(table.at[idx_v.at[g]], bufs[k], gsems[k])
            for k in range(nring):
                g = nring * gq + k
                pltpu.make_async_copy(table.at[idx_v.at[g]], bufs[k],
                                      gsems[k]).wait()
                pltpu.async_copy(bufs[k], out.at[base + g], wsems[k])
            return carry

        lax.fori_loop(0, nq, body, 0)
        for k in range(nring):
            pltpu.make_async_copy(bufs[k], out.at[base], wsems[k]).wait()

        def tail(t, carry):
            g = nring * nq + t

            @pl.when(g < nb)
            def _():
                pltpu.async_copy(table.at[idx_v.at[g]], bufs[0],
                                 gsems[0]).wait()
                pltpu.sync_copy(bufs[0], out.at[base + g])

            return carry

        lax.fori_loop(0, nring - 1, tail, 0)

    nb0 = jnp.where(last, _NB0 - (_NW - 1) * _BPW0, _BPW0)
    base0 = wid * _BPW0
    pltpu.sync_copy(idx0.at[pl.ds(base0, _BPW0)], i0_v)
    run_pass(f0t, i0_v, out0, base0, nb0)

    nb1 = jnp.where(last, _NB1 - (_NW - 1) * _BPW1, _BPW1)
    base1 = wid * _BPW1
    pltpu.sync_copy(idx1.at[pl.ds(wid * _STRIDE1, _STRIDE1)], i1_v)
    run_pass(f1t, i1_v, out1, base1, nb1)


@jax.jit
def kernel(feature0, feature1, b_idxes, i_idxes, j_idxes):
    f0t = jnp.pad(jnp.transpose(feature0, (0, 2, 3, 1)),
                  ((0, 0), (_PAD, _PAD), (_PAD, _W0P - _W - _PAD), (0, 0)))
    f0t = f0t.reshape(_B * _H0 * _W0P, _C)
    f1t = jnp.pad(jnp.transpose(feature1, (0, 2, 3, 1)),
                  ((0, 0), (_PAD + _EXTRA, _PAD + _EXTRA),
                   (_PAD + _EXTRA, _PAD + _EXTRA), (0, 0)))
    f1t = f1t.reshape(_B * _H1 * _H1, _C)

    b = b_idxes.astype(jnp.int32)
    i = i_idxes.astype(jnp.int32)
    j = j_idxes.astype(jnp.int32)

    r0 = (i // _GRID) * _STRIDE
    c0 = (i % _GRID) * _STRIDE
    p0 = jnp.arange(_PIX0, dtype=jnp.int32)
    idx0 = b[:, None] * (_H0 * _W0P) \
        + (r0[:, None] + p0[None, :] // _W_SIZE) * _W0P \
        + c0[:, None] + p0[None, :] % _W_SIZE
    idx0 = idx0.reshape(_NB0, 128)
    idx0 = jnp.pad(idx0, ((0, _NW * _BPW0 - _NB0), (0, 0)))

    r1 = (j // _GRID) * _STRIDE
    c1 = (j % _GRID) * _STRIDE
    p1 = jnp.arange(_PIX1, dtype=jnp.int32)
    idx1 = b[:, None] * (_H1 * _H1) \
        + (r1[:, None] + p1[None, :] // _K1) * _H1 \
        + c1[:, None] + p1[None, :] % _K1
    idx1 = idx1.reshape(_NB1, 128)
    idx1 = jnp.pad(idx1, ((0, _NW * _BPW1 - _NB1), (0, 0)))
    idx1 = jnp.pad(idx1.reshape(_NW, _BPW1, 128),
                   ((0, 0), (0, _STRIDE1 - _BPW1), (0, 0)))
    idx1 = idx1.reshape(_NW * _STRIDE1, 128)

    mesh = plsc.VectorSubcoreMesh(core_axis_name="c", subcore_axis_name="s")
    out0, out1 = pl.kernel(
        _gather_kernel,
        mesh=mesh,
        out_type=[
            jax.ShapeDtypeStruct((_NB0, 128, _C), jnp.float32),
            jax.ShapeDtypeStruct((_NB1, 128, _C), jnp.float32),
        ],
        scratch_types=[
            pltpu.VMEM((_BPW0, 128), jnp.int32),
            pltpu.VMEM((_STRIDE1, 128), jnp.int32),
            pltpu.VMEM((128, _C), jnp.float32),
            pltpu.VMEM((128, _C), jnp.float32),
            pltpu.VMEM((128, _C), jnp.float32),
            pltpu.VMEM((128, _C), jnp.float32),
            pltpu.SemaphoreType.DMA,
            pltpu.SemaphoreType.DMA,
            pltpu.SemaphoreType.DMA,
            pltpu.SemaphoreType.DMA,
            pltpu.SemaphoreType.DMA,
            pltpu.SemaphoreType.DMA,
            pltpu.SemaphoreType.DMA,
            pltpu.SemaphoreType.DMA,
        ],
    )(f0t, f1t, idx0, idx1)

    fine0 = out0.reshape(_M, _PIX0, _C)
    fine1 = out1.reshape(_M, _PIX1, _C)
    return (fine0, fine1)


# speedup vs baseline: 5.0720x; 1.0026x over previous
import functools
import jax
import jax.numpy as jnp
from jax import lax
from jax.experimental import pallas as pl
from jax.experimental.pallas import tpu as pltpu
from jax.experimental.pallas import tpu_sc as plsc

_W_SIZE = 8
_STRIDE = 4
_PAD = 2
_EXTRA = 2

_B, _C, _H, _W = 2, 128, 192, 192
_GRID = (_H + 2 * _PAD - _W_SIZE) // _STRIDE + 1
_M = 3000

_H0 = _H + 2 * _PAD
_W0P = 200
_PIX0 = _W_SIZE * _W_SIZE
_H1 = _H + 2 * (_PAD + _EXTRA)
_K1 = _W_SIZE + 2 * _EXTRA
_PIX1 = _K1 * _K1

_NW = 32
_NB0 = _M * _PIX0 // 128
_NB1 = _M * _PIX1 // 128
_BPW0 = 48
_BPW1 = 108
_STRIDE1 = 112


def _gather_kernel(f0t, f1t, idx0, idx1, out0, out1, i0_v, i1_v,
                   b0, b1, b2, b3, b4, b5, gs0, gs1, gs2, gs3, gs4, gs5,
                   ws0, ws1, ws2, ws3, ws4, ws5):
    bufs = (b0, b1, b2, b3, b4, b5)
    gsems = (gs0, gs1, gs2, gs3, gs4, gs5)
    wsems = (ws0, ws1, ws2, ws3, ws4, ws5)
    wid = lax.axis_index("c") * 16 + lax.axis_index("s")
    last = wid == _NW - 1
    nring = len(bufs)

    def run_pass(table, idx_v, out, base, nb):
        nq = nb // nring

        def body(gq, carry):
            for k in range(nring):
                g = nring * gq + k

                @pl.when(gq > 0)
                def _(k=k):
                    pltpu.make_async_copy(bufs[k], out.at[base],
                                          wsems[k]).wait()

                pltpu.async_copy(table.at[idx_v.at[g]], bufs[k], gsems[k])
            for k in range(nring):
                g = nring * gq + k
                pltpu.make_async_copy(table.at[idx_v.at[g]], bufs[k],
                                      gsems[k]).wait()
                pltpu.async_copy(bufs[k], out.at[base + g], wsems[k])
            return carry

        lax.fori_loop(0, nq, body, 0)
        for k in range(nring):
            pltpu.make_async_copy(bufs[k], out.at[base], wsems[k]).wait()

        def tail(t, carry):
            g = nring * nq + t

            @pl.when(g < nb)
            def _():
                pltpu.async_copy(table.at[idx_v.at[g]], bufs[0],
                                 gsems[0]).wait()
                pltpu.sync_copy(bufs[0], out.at[base + g])

            return carry

        lax.fori_loop(0, nring - 1, tail, 0)

    nb0 = jnp.where(last, _NB0 - (_NW - 1) * _BPW0, _BPW0)
    base0 = wid * _BPW0
    pltpu.sync_copy(idx0.at[pl.ds(base0, _BPW0)], i0_v)
    run_pass(f0t, i0_v, out0, base0, nb0)

    nb1 = jnp.where(last, _NB1 - (_NW - 1) * _BPW1, _BPW1)
    base1 = wid * _BPW1
    pltpu.sync_copy(idx1.at[pl.ds(wid * _STRIDE1, _STRIDE1)], i1_v)
    run_pass(f1t, i1_v, out1, base1, nb1)


@jax.jit
def kernel(feature0, feature1, b_idxes, i_idxes, j_idxes):
    f0t = jnp.pad(jnp.transpose(feature0, (0, 2, 3, 1)),
                  ((0, 0), (_PAD, _PAD), (_PAD, _W0P - _W - _PAD), (0, 0)))
    f0t = f0t.reshape(_B * _H0 * _W0P, _C)
    f1t = jnp.pad(jnp.transpose(feature1, (0, 2, 3, 1)),
                  ((0, 0), (_PAD + _EXTRA, _PAD + _EXTRA),
                   (_PAD + _EXTRA, _PAD + _EXTRA), (0, 0)))
    f1t = f1t.reshape(_B * _H1 * _H1, _C)

    b = b_idxes.astype(jnp.int32)
    i = i_idxes.astype(jnp.int32)
    j = j_idxes.astype(jnp.int32)

    r0 = (i // _GRID) * _STRIDE
    c0 = (i % _GRID) * _STRIDE
    p0 = jnp.arange(_PIX0, dtype=jnp.int32)
    idx0 = b[:, None] * (_H0 * _W0P) \
        + (r0[:, None] + p0[None, :] // _W_SIZE) * _W0P \
        + c0[:, None] + p0[None, :] % _W_SIZE
    idx0 = idx0.reshape(_NB0, 128)
    idx0 = jnp.pad(idx0, ((0, _NW * _BPW0 - _NB0), (0, 0)))

    r1 = (j // _GRID) * _STRIDE
    c1 = (j % _GRID) * _STRIDE
    p1 = jnp.arange(_PIX1, dtype=jnp.int32)
    idx1 = b[:, None] * (_H1 * _H1) \
        + (r1[:, None] + p1[None, :] // _K1) * _H1 \
        + c1[:, None] + p1[None, :] % _K1
    idx1 = idx1.reshape(_NB1, 128)
    idx1 = jnp.pad(idx1, ((0, _NW * _BPW1 - _NB1), (0, 0)))
    idx1 = jnp.pad(idx1.reshape(_NW, _BPW1, 128),
                   ((0, 0), (0, _STRIDE1 - _BPW1), (0, 0)))
    idx1 = idx1.reshape(_NW * _STRIDE1, 128)

    mesh = plsc.VectorSubcoreMesh(core_axis_name="c", subcore_axis_name="s")
    out0, out1 = pl.kernel(
        _gather_kernel,
        mesh=mesh,
        out_type=[
            jax.ShapeDtypeStruct((_NB0, 128, _C), jnp.float32),
            jax.ShapeDtypeStruct((_NB1, 128, _C), jnp.float32),
        ],
        scratch_types=[
            pltpu.VMEM((_BPW0, 128), jnp.int32),
            pltpu.VMEM((_STRIDE1, 128), jnp.int32),
            pltpu.VMEM((128, _C), jnp.float32),
            pltpu.VMEM((128, _C), jnp.float32),
            pltpu.VMEM((128, _C), jnp.float32),
            pltpu.VMEM((128, _C), jnp.float32),
            pltpu.VMEM((128, _C), jnp.float32),
            pltpu.VMEM((128, _C), jnp.float32),
            pltpu.SemaphoreType.DMA,
            pltpu.SemaphoreType.DMA,
            pltpu.SemaphoreType.DMA,
            pltpu.SemaphoreType.DMA,
            pltpu.SemaphoreType.DMA,
            pltpu.SemaphoreType.DMA,
            pltpu.SemaphoreType.DMA,
            pltpu.SemaphoreType.DMA,
            pltpu.SemaphoreType.DMA,
            pltpu.SemaphoreType.DMA,
            pltpu.SemaphoreType.DMA,
            pltpu.SemaphoreType.DMA,
        ],
    )(f0t, f1t, idx0, idx1)

    fine0 = out0.reshape(_M, _PIX0, _C)
    fine1 = out1.reshape(_M, _PIX1, _C)
    return (fine0, fine1)
